# baseline (speedup 1.0000x reference)
import jax
import jax.numpy as jnp
from jax.experimental import pallas as pl
from jax.experimental.pallas import tpu as pltpu

_B, _N, _F, _C = 8, 2048, 512, 32
_BLK = 256
_NB = _N // _BLK


def _score_kernel(x1_ref, x2_ref, wt_ref, idx1_ref, idx2_ref,
                  m1_s, i1_s, m2_s, i2_s):
    nb = pl.program_id(1)
    wt = wt_ref[...]

    @pl.when(nb == 0)
    def _init():
        m1_s[...] = jnp.full((1, _C), -jnp.inf, jnp.float32)
        m2_s[...] = jnp.full((1, _C), -jnp.inf, jnp.float32)
        i1_s[...] = jnp.zeros((1, _C), jnp.int32)
        i2_s[...] = jnp.zeros((1, _C), jnp.int32)

    iota = jax.lax.broadcasted_iota(jnp.int32, (_BLK, _C), 0)
    for x_ref, m_s, i_s in ((x1_ref, m1_s, i1_s), (x2_ref, m2_s, i2_s)):
        x = x_ref[0]
        scores = jnp.dot(x, wt, preferred_element_type=jnp.float32)
        bmax = jnp.max(scores, axis=0, keepdims=True)
        bidx = jnp.min(
            jnp.where(scores == bmax, iota, _BLK), axis=0, keepdims=True
        ) + nb * _BLK
        better = bmax > m_s[...]
        i_s[...] = jnp.where(better, bidx, i_s[...])
        m_s[...] = jnp.where(better, bmax, m_s[...])

    @pl.when(nb == _NB - 1)
    def _emit():
        idx1_ref[0, 0, :] = i1_s[0, :]
        idx2_ref[0, 0, :] = i2_s[0, :]


def _gather_kernel(i1_ref, i2_ref, x1_ref, x2_ref, d_ref, d1_ref, sem):
    copies = []
    for b in range(_B):
        for c in range(_C):
            r1 = i1_ref[b, 0, c]
            r2 = i2_ref[b, 0, c]
            cp1 = pltpu.make_async_copy(
                x1_ref.at[b, pl.ds(r1, 1), :], d_ref.at[b, pl.ds(c, 1), :], sem)
            cp2 = pltpu.make_async_copy(
                x2_ref.at[b, pl.ds(r2, 1), :], d1_ref.at[b, pl.ds(c, 1), :], sem)
            cp1.start()
            cp2.start()
            copies.append(cp1)
            copies.append(cp2)
    for cp in copies:
        cp.wait()


def kernel(x1, x2, W, b):
    del b
    wt = W.T
    idx1, idx2 = pl.pallas_call(
        _score_kernel,
        grid=(_B, _NB),
        in_specs=[
            pl.BlockSpec((1, _BLK, _F), lambda i, j: (i, j, 0)),
            pl.BlockSpec((1, _BLK, _F), lambda i, j: (i, j, 0)),
            pl.BlockSpec((_F, _C), lambda i, j: (0, 0)),
        ],
        out_specs=[
            pl.BlockSpec((1, 1, _C), lambda i, j: (i, 0, 0)),
            pl.BlockSpec((1, 1, _C), lambda i, j: (i, 0, 0)),
        ],
        out_shape=[
            jax.ShapeDtypeStruct((_B, 1, _C), jnp.int32),
            jax.ShapeDtypeStruct((_B, 1, _C), jnp.int32),
        ],
        scratch_shapes=[
            pltpu.VMEM((1, _C), jnp.float32),
            pltpu.VMEM((1, _C), jnp.int32),
            pltpu.VMEM((1, _C), jnp.float32),
            pltpu.VMEM((1, _C), jnp.int32),
        ],
    )(x1, x2, wt)

    d, d1 = pl.pallas_call(
        _gather_kernel,
        grid_spec=pltpu.PrefetchScalarGridSpec(
            num_scalar_prefetch=2,
            grid=(1,),
            in_specs=[
                pl.BlockSpec(memory_space=pl.ANY),
                pl.BlockSpec(memory_space=pl.ANY),
            ],
            out_specs=[
                pl.BlockSpec((_B, _C, _F), lambda i, i1, i2: (0, 0, 0)),
                pl.BlockSpec((_B, _C, _F), lambda i, i1, i2: (0, 0, 0)),
            ],
            scratch_shapes=[pltpu.SemaphoreType.DMA],
        ),
        out_shape=[
            jax.ShapeDtypeStruct((_B, _C, _F), jnp.float32),
            jax.ShapeDtypeStruct((_B, _C, _F), jnp.float32),
        ],
    )(idx1, idx2, x1, x2)
    return (d, d1)


# speedup vs baseline: 1.3155x; 1.3155x over previous
import jax
import jax.numpy as jnp
from jax.experimental import pallas as pl
from jax.experimental.pallas import tpu as pltpu

_B, _N, _F, _C = 8, 2048, 512, 32
_BLK = 512
_NB = _N // _BLK


def _score_kernel(x1_ref, x2_ref, w_ref, idx1_ref, idx2_ref,
                  m1_s, i1_s, m2_s, i2_s):
    nb = pl.program_id(1)
    w = w_ref[...]

    @pl.when(nb == 0)
    def _init():
        m1_s[...] = jnp.full((_C, 1), -jnp.inf, jnp.float32)
        m2_s[...] = jnp.full((_C, 1), -jnp.inf, jnp.float32)
        i1_s[...] = jnp.zeros((_C, 1), jnp.int32)
        i2_s[...] = jnp.zeros((_C, 1), jnp.int32)

    iota = jax.lax.broadcasted_iota(jnp.int32, (_C, _BLK), 1)
    for x_ref, m_s, i_s in ((x1_ref, m1_s, i1_s), (x2_ref, m2_s, i2_s)):
        x = x_ref[0]
        scores_t = jax.lax.dot_general(
            w, x, (((1,), (1,)), ((), ())),
            preferred_element_type=jnp.float32,
        )
        bmax = jnp.max(scores_t, axis=1, keepdims=True)
        bidx = jnp.min(
            jnp.where(scores_t == bmax, iota, _BLK), axis=1, keepdims=True
        ) + nb * _BLK
        better = bmax > m_s[...]
        i_s[...] = jnp.where(better, bidx, i_s[...])
        m_s[...] = jnp.where(better, bmax, m_s[...])

    @pl.when(nb == _NB - 1)
    def _emit():
        idx1_ref[0] = i1_s[...]
        idx2_ref[0] = i2_s[...]


def _gather_kernel(i1_ref, i2_ref, x1_ref, x2_ref, d_ref, d1_ref, sem):
    copies = []
    for b in range(_B):
        for c in range(_C):
            r1 = i1_ref[b, c, 0]
            r2 = i2_ref[b, c, 0]
            cp1 = pltpu.make_async_copy(
                x1_ref.at[b, pl.ds(r1, 1), :], d_ref.at[b, pl.ds(c, 1), :], sem)
            cp2 = pltpu.make_async_copy(
                x2_ref.at[b, pl.ds(r2, 1), :], d1_ref.at[b, pl.ds(c, 1), :], sem)
            cp1.start()
            cp2.start()
            copies.append(cp1)
            copies.append(cp2)
    for cp in copies:
        cp.wait()


def kernel(x1, x2, W, b):
    del b
    idx1, idx2 = pl.pallas_call(
        _score_kernel,
        grid=(_B, _NB),
        in_specs=[
            pl.BlockSpec((1, _BLK, _F), lambda i, j: (i, j, 0)),
            pl.BlockSpec((1, _BLK, _F), lambda i, j: (i, j, 0)),
            pl.BlockSpec((_C, _F), lambda i, j: (0, 0)),
        ],
        out_specs=[
            pl.BlockSpec((1, _C, 1), lambda i, j: (i, 0, 0)),
            pl.BlockSpec((1, _C, 1), lambda i, j: (i, 0, 0)),
        ],
        out_shape=[
            jax.ShapeDtypeStruct((_B, _C, 1), jnp.int32),
            jax.ShapeDtypeStruct((_B, _C, 1), jnp.int32),
        ],
        scratch_shapes=[
            pltpu.VMEM((_C, 1), jnp.float32),
            pltpu.VMEM((_C, 1), jnp.int32),
            pltpu.VMEM((_C, 1), jnp.float32),
            pltpu.VMEM((_C, 1), jnp.int32),
        ],
    )(x1, x2, W)

    d, d1 = pl.pallas_call(
        _gather_kernel,
        grid_spec=pltpu.PrefetchScalarGridSpec(
            num_scalar_prefetch=2,
            grid=(1,),
            in_specs=[
                pl.BlockSpec(memory_space=pl.ANY),
                pl.BlockSpec(memory_space=pl.ANY),
            ],
            out_specs=[
                pl.BlockSpec((_B, _C, _F), lambda i, i1, i2: (0, 0, 0)),
                pl.BlockSpec((_B, _C, _F), lambda i, i1, i2: (0, 0, 0)),
            ],
            scratch_shapes=[pltpu.SemaphoreType.DMA],
        ),
        out_shape=[
            jax.ShapeDtypeStruct((_B, _C, _F), jnp.float32),
            jax.ShapeDtypeStruct((_B, _C, _F), jnp.float32),
        ],
    )(idx1, idx2, x1, x2)
    return (d, d1)


# speedup vs baseline: 1.6794x; 1.2766x over previous
import jax
import jax.numpy as jnp
from jax.experimental import pallas as pl
from jax.experimental.pallas import tpu as pltpu

_B, _N, _F, _C = 8, 2048, 512, 32
_BLK = 1024
_NB = _N // _BLK


def _score_kernel(x1_ref, x2_ref, w_ref, idx1_ref, idx2_ref,
                  m1_s, i1_s, m2_s, i2_s):
    nb = pl.program_id(1)
    w = w_ref[...]

    @pl.when(nb == 0)
    def _init():
        m1_s[...] = jnp.full((_C, 1), -jnp.inf, jnp.float32)
        m2_s[...] = jnp.full((_C, 1), -jnp.inf, jnp.float32)
        i1_s[...] = jnp.zeros((_C, 1), jnp.int32)
        i2_s[...] = jnp.zeros((_C, 1), jnp.int32)

    iota = jax.lax.broadcasted_iota(jnp.int32, (_C, _BLK), 1)
    for x_ref, m_s, i_s in ((x1_ref, m1_s, i1_s), (x2_ref, m2_s, i2_s)):
        x = x_ref[0]
        scores_t = jax.lax.dot_general(
            w, x, (((1,), (1,)), ((), ())),
            preferred_element_type=jnp.float32,
        )
        bmax = jnp.max(scores_t, axis=1, keepdims=True)
        bidx = jnp.min(
            jnp.where(scores_t == bmax, iota, _BLK), axis=1, keepdims=True
        ) + nb * _BLK
        better = bmax > m_s[...]
        i_s[...] = jnp.where(better, bidx, i_s[...])
        m_s[...] = jnp.where(better, bmax, m_s[...])

    @pl.when(nb == _NB - 1)
    def _emit():
        idx1_ref[0] = i1_s[...]
        idx2_ref[0] = i2_s[...]


def _gather_kernel(i1_ref, i2_ref, x1_ref, x2_ref, d_ref, d1_ref, sem):
    copies = []
    for b in range(_B):
        for c in range(_C):
            r1 = i1_ref[b, c, 0]
            r2 = i2_ref[b, c, 0]
            cp1 = pltpu.make_async_copy(
                x1_ref.at[b, pl.ds(r1, 1), :], d_ref.at[b, pl.ds(c, 1), :], sem)
            cp2 = pltpu.make_async_copy(
                x2_ref.at[b, pl.ds(r2, 1), :], d1_ref.at[b, pl.ds(c, 1), :], sem)
            cp1.start()
            cp2.start()
            copies.append(cp1)
            copies.append(cp2)
    for cp in copies:
        cp.wait()


def kernel(x1, x2, W, b):
    del b
    idx1, idx2 = pl.pallas_call(
        _score_kernel,
        grid=(_B, _NB),
        in_specs=[
            pl.BlockSpec((1, _BLK, _F), lambda i, j: (i, j, 0)),
            pl.BlockSpec((1, _BLK, _F), lambda i, j: (i, j, 0)),
            pl.BlockSpec((_C, _F), lambda i, j: (0, 0)),
        ],
        out_specs=[
            pl.BlockSpec((1, _C, 1), lambda i, j: (i, 0, 0)),
            pl.BlockSpec((1, _C, 1), lambda i, j: (i, 0, 0)),
        ],
        out_shape=[
            jax.ShapeDtypeStruct((_B, _C, 1), jnp.int32),
            jax.ShapeDtypeStruct((_B, _C, 1), jnp.int32),
        ],
        scratch_shapes=[
            pltpu.VMEM((_C, 1), jnp.float32),
            pltpu.VMEM((_C, 1), jnp.int32),
            pltpu.VMEM((_C, 1), jnp.float32),
            pltpu.VMEM((_C, 1), jnp.int32),
        ],
    )(x1, x2, W)

    d, d1 = pl.pallas_call(
        _gather_kernel,
        grid_spec=pltpu.PrefetchScalarGridSpec(
            num_scalar_prefetch=2,
            grid=(1,),
            in_specs=[
                pl.BlockSpec(memory_space=pl.ANY),
                pl.BlockSpec(memory_space=pl.ANY),
            ],
            out_specs=[
                pl.BlockSpec((_B, _C, _F), lambda i, i1, i2: (0, 0, 0)),
                pl.BlockSpec((_B, _C, _F), lambda i, i1, i2: (0, 0, 0)),
            ],
            scratch_shapes=[pltpu.SemaphoreType.DMA],
        ),
        out_shape=[
            jax.ShapeDtypeStruct((_B, _C, _F), jnp.float32),
            jax.ShapeDtypeStruct((_B, _C, _F), jnp.float32),
        ],
    )(idx1, idx2, x1, x2)
    return (d, d1)


# speedup vs baseline: 1.9151x; 1.1404x over previous
import jax
import jax.numpy as jnp
from jax.experimental import pallas as pl
from jax.experimental.pallas import tpu as pltpu

_B, _N, _F, _C = 8, 2048, 512, 32
_BLK = 2048
_NB = _N // _BLK


def _score_kernel(x1_ref, x2_ref, w_ref, idx1_ref, idx2_ref,
                  m1_s, i1_s, m2_s, i2_s):
    nb = pl.program_id(1)
    w = w_ref[...]

    @pl.when(nb == 0)
    def _init():
        m1_s[...] = jnp.full((_C, 1), -jnp.inf, jnp.float32)
        m2_s[...] = jnp.full((_C, 1), -jnp.inf, jnp.float32)
        i1_s[...] = jnp.zeros((_C, 1), jnp.int32)
        i2_s[...] = jnp.zeros((_C, 1), jnp.int32)

    iota = jax.lax.broadcasted_iota(jnp.int32, (_C, _BLK), 1)
    for x_ref, m_s, i_s in ((x1_ref, m1_s, i1_s), (x2_ref, m2_s, i2_s)):
        x = x_ref[0]
        scores_t = jax.lax.dot_general(
            w, x, (((1,), (1,)), ((), ())),
            preferred_element_type=jnp.float32,
        )
        bmax = jnp.max(scores_t, axis=1, keepdims=True)
        bidx = jnp.min(
            jnp.where(scores_t == bmax, iota, _BLK), axis=1, keepdims=True
        ) + nb * _BLK
        better = bmax > m_s[...]
        i_s[...] = jnp.where(better, bidx, i_s[...])
        m_s[...] = jnp.where(better, bmax, m_s[...])

    @pl.when(nb == _NB - 1)
    def _emit():
        idx1_ref[0] = i1_s[...]
        idx2_ref[0] = i2_s[...]


def _gather_kernel(i1_ref, i2_ref, x1_ref, x2_ref, d_ref, d1_ref, sem):
    copies = []
    for b in range(_B):
        for c in range(_C):
            r1 = i1_ref[b, c, 0]
            r2 = i2_ref[b, c, 0]
            cp1 = pltpu.make_async_copy(
                x1_ref.at[b, pl.ds(r1, 1), :], d_ref.at[b, pl.ds(c, 1), :], sem)
            cp2 = pltpu.make_async_copy(
                x2_ref.at[b, pl.ds(r2, 1), :], d1_ref.at[b, pl.ds(c, 1), :], sem)
            cp1.start()
            cp2.start()
            copies.append(cp1)
            copies.append(cp2)
    for cp in copies:
        cp.wait()


def kernel(x1, x2, W, b):
    del b
    idx1, idx2 = pl.pallas_call(
        _score_kernel,
        grid=(_B, _NB),
        in_specs=[
            pl.BlockSpec((1, _BLK, _F), lambda i, j: (i, j, 0)),
            pl.BlockSpec((1, _BLK, _F), lambda i, j: (i, j, 0)),
            pl.BlockSpec((_C, _F), lambda i, j: (0, 0)),
        ],
        out_specs=[
            pl.BlockSpec((1, _C, 1), lambda i, j: (i, 0, 0)),
            pl.BlockSpec((1, _C, 1), lambda i, j: (i, 0, 0)),
        ],
        out_shape=[
            jax.ShapeDtypeStruct((_B, _C, 1), jnp.int32),
            jax.ShapeDtypeStruct((_B, _C, 1), jnp.int32),
        ],
        scratch_shapes=[
            pltpu.VMEM((_C, 1), jnp.float32),
            pltpu.VMEM((_C, 1), jnp.int32),
            pltpu.VMEM((_C, 1), jnp.float32),
            pltpu.VMEM((_C, 1), jnp.int32),
        ],
    )(x1, x2, W)

    d, d1 = pl.pallas_call(
        _gather_kernel,
        grid_spec=pltpu.PrefetchScalarGridSpec(
            num_scalar_prefetch=2,
            grid=(1,),
            in_specs=[
                pl.BlockSpec(memory_space=pl.ANY),
                pl.BlockSpec(memory_space=pl.ANY),
            ],
            out_specs=[
                pl.BlockSpec((_B, _C, _F), lambda i, i1, i2: (0, 0, 0)),
                pl.BlockSpec((_B, _C, _F), lambda i, i1, i2: (0, 0, 0)),
            ],
            scratch_shapes=[pltpu.SemaphoreType.DMA],
        ),
        out_shape=[
            jax.ShapeDtypeStruct((_B, _C, _F), jnp.float32),
            jax.ShapeDtypeStruct((_B, _C, _F), jnp.float32),
        ],
    )(idx1, idx2, x1, x2)
    return (d, d1)


# speedup vs baseline: 1.9173x; 1.0011x over previous
import jax
import jax.numpy as jnp
from jax.experimental import pallas as pl
from jax.experimental.pallas import tpu as pltpu

_B, _N, _F, _C = 8, 2048, 512, 32
_NSPLIT = 2
_BLK = _N // _NSPLIT


def _half_argmax(w, x, iota, base):
    scores_t = jax.lax.dot_general(
        w, x, (((1,), (1,)), ((), ())),
        preferred_element_type=jnp.float32,
    )
    hmax = jnp.max(scores_t, axis=1, keepdims=True)
    hidx = jnp.min(
        jnp.where(scores_t == hmax, iota, _BLK), axis=1, keepdims=True
    ) + base
    return hmax, hidx


def _score_kernel(x1a_ref, x1b_ref, x2a_ref, x2b_ref, w_ref,
                  idx1_ref, idx2_ref):
    w = w_ref[...]
    iota = jax.lax.broadcasted_iota(jnp.int32, (_C, _BLK), 1)
    for (a_ref, b_ref), idx_ref in (((x1a_ref, x1b_ref), idx1_ref),
                                    ((x2a_ref, x2b_ref), idx2_ref)):
        amax, aidx = _half_argmax(w, a_ref[0], iota, 0)
        bmax, bidx = _half_argmax(w, b_ref[0], iota, _BLK)
        better = bmax > amax
        idx_ref[0] = jnp.where(better, bidx, aidx)


def _gather_kernel(i1_ref, i2_ref, x1_ref, x2_ref, d_ref, d1_ref, sem):
    copies = []
    for b in range(_B):
        for c in range(_C):
            r1 = i1_ref[b, c, 0]
            r2 = i2_ref[b, c, 0]
            cp1 = pltpu.make_async_copy(
                x1_ref.at[b, pl.ds(r1, 1), :], d_ref.at[b, pl.ds(c, 1), :], sem)
            cp2 = pltpu.make_async_copy(
                x2_ref.at[b, pl.ds(r2, 1), :], d1_ref.at[b, pl.ds(c, 1), :], sem)
            cp1.start()
            cp2.start()
            copies.append(cp1)
            copies.append(cp2)
    for cp in copies:
        cp.wait()


def kernel(x1, x2, W, b):
    del b
    idx1, idx2 = pl.pallas_call(
        _score_kernel,
        grid=(_B,),
        in_specs=[
            pl.BlockSpec((1, _BLK, _F), lambda i: (i, 0, 0)),
            pl.BlockSpec((1, _BLK, _F), lambda i: (i, 1, 0)),
            pl.BlockSpec((1, _BLK, _F), lambda i: (i, 0, 0)),
            pl.BlockSpec((1, _BLK, _F), lambda i: (i, 1, 0)),
            pl.BlockSpec((_C, _F), lambda i: (0, 0)),
        ],
        out_specs=[
            pl.BlockSpec((1, _C, 1), lambda i: (i, 0, 0)),
            pl.BlockSpec((1, _C, 1), lambda i: (i, 0, 0)),
        ],
        out_shape=[
            jax.ShapeDtypeStruct((_B, _C, 1), jnp.int32),
            jax.ShapeDtypeStruct((_B, _C, 1), jnp.int32),
        ],
    )(x1, x1, x2, x2, W)

    d, d1 = pl.pallas_call(
        _gather_kernel,
        grid_spec=pltpu.PrefetchScalarGridSpec(
            num_scalar_prefetch=2,
            grid=(1,),
            in_specs=[
                pl.BlockSpec(memory_space=pl.ANY),
                pl.BlockSpec(memory_space=pl.ANY),
            ],
            out_specs=[
                pl.BlockSpec((_B, _C, _F), lambda i, i1, i2: (0, 0, 0)),
                pl.BlockSpec((_B, _C, _F), lambda i, i1, i2: (0, 0, 0)),
            ],
            scratch_shapes=[pltpu.SemaphoreType.DMA],
        ),
        out_shape=[
            jax.ShapeDtypeStruct((_B, _C, _F), jnp.float32),
            jax.ShapeDtypeStruct((_B, _C, _F), jnp.float32),
        ],
    )(idx1, idx2, x1, x2)
    return (d, d1)
